# baseline (speedup 1.0000x reference)
import functools

import jax
import jax.numpy as jnp
import numpy as np
from jax import lax
from jax.experimental import pallas as pl
from jax.experimental.pallas import tpu as pltpu
from jax.experimental.pallas import tpu_sc as plsc

NC = 2
NS = 16
NW = NC * NS
LANES = 16

SEG_BATCH = 32


def _sc_weighted_segsum(key_flat, table_half, pe_half, B, M, L):
    S = B * M
    Dh = table_half.shape[1]
    segs_per_w = S // NW
    n_batches = segs_per_w // SEG_BATCH
    rows_per_batch = SEG_BATCH * L
    idx_chunks = rows_per_batch // 128
    idx_per_w = n_batches * idx_chunks * 128

    mesh = plsc.VectorSubcoreMesh(core_axis_name="c", subcore_axis_name="s")

    @functools.partial(
        pl.kernel,
        out_type=jax.ShapeDtypeStruct((S, Dh), jnp.float32),
        mesh=mesh,
        scratch_types=[
            pltpu.VMEM((idx_per_w,), jnp.int32),
            pltpu.VMEM((L, Dh), jnp.float32),
            pltpu.VMEM((2, rows_per_batch, Dh), jnp.bfloat16),
            pltpu.VMEM((2, SEG_BATCH, Dh), jnp.float32),
            pltpu.VMEM((2, SEG_BATCH), jnp.int32),
            pltpu.SemaphoreType.DMA,
            pltpu.SemaphoreType.DMA,
            pltpu.SemaphoreType.DMA,
            pltpu.SemaphoreType.DMA,
        ],
        compiler_params=pltpu.CompilerParams(
            use_tc_tiling_on_sc=False, needs_layout_passes=False
        ),
    )
    def k(key_hbm, table_hbm, pe_hbm, out_hbm, idx_v, pe_v, rows_v, out_v,
          oidx_v, sem0, sem1, osem0, osem1):
        wid = lax.axis_index("s") * NC + lax.axis_index("c")
        pltpu.sync_copy(key_hbm.at[pl.ds(wid * idx_per_w, idx_per_w)], idx_v)
        pltpu.sync_copy(pe_hbm, pe_v)
        sems = (sem0, sem1)
        osems = (osem0, osem1)
        iota16 = lax.iota(jnp.int32, LANES)

        def fire(b, slot):
            for j in range(idx_chunks):
                pltpu.async_copy(
                    table_hbm.at[idx_v.at[pl.ds((b * idx_chunks + j) * 128, 128)]],
                    rows_v.at[slot].at[pl.ds(j * 128, 128)],
                    sems[slot],
                )

        def drain(slot):
            pltpu.make_async_copy(
                table_hbm.at[pl.ds(0, rows_per_batch)],
                rows_v.at[slot],
                sems[slot],
            ).wait()

        def drain_out(slot):
            pltpu.make_async_copy(
                out_v.at[slot],
                out_hbm.at[pl.ds(0, SEG_BATCH)],
                osems[slot],
            ).wait()

        def compute(b, slot):
            @pl.when(b >= 2)
            def _(slot=slot):
                drain_out(slot)

            s0 = wid * segs_per_w + b * SEG_BATCH
            for h in range(SEG_BATCH // LANES):
                sv = iota16 + (s0 + h * LANES)
                rv = (sv % M) * B + sv // M
                oidx_v[slot, pl.ds(h * LANES, LANES)] = rv

            sl32 = pl.ds(0, Dh)
            pe_e = [pe_v[l, pl.ds(0, LANES)] for l in range(L)]
            pe_o = [pe_v[l, pl.ds(LANES, LANES)] for l in range(L)]

            def seg_body(s, _, sl32=sl32, pe_e=pe_e, pe_o=pe_o, slot=slot):
                base = s * L
                packed = rows_v[slot, base, sl32]
                ev, od = plsc.unpack(
                    packed,
                    format=plsc.PackFormat.INTERLEAVED,
                    preferred_element_type=jnp.float32,
                )
                acc_e = pe_e[0] * ev
                acc_o = pe_o[0] * od
                for l in range(1, L):
                    packed = rows_v[slot, base + l, sl32]
                    ev, od = plsc.unpack(
                        packed,
                        format=plsc.PackFormat.INTERLEAVED,
                        preferred_element_type=jnp.float32,
                    )
                    acc_e = acc_e + pe_e[l] * ev
                    acc_o = acc_o + pe_o[l] * od
                out_v[slot, s, pl.ds(0, LANES)] = acc_e
                out_v[slot, s, pl.ds(LANES, LANES)] = acc_o
                return 0

            lax.fori_loop(0, SEG_BATCH, seg_body, 0)

            pltpu.async_copy(
                out_v.at[slot], out_hbm.at[oidx_v.at[slot]], osems[slot]
            )

        fire(0, 0)
        fire(1, 1)

        def pair_body(i, carry):
            b = i * 2
            for slot in range(2):
                drain(slot)
                compute(b + slot, slot)

                @pl.when(b + slot + 2 < n_batches)
                def _(b=b, slot=slot):
                    fire(b + slot + 2, slot)

            return carry

        lax.fori_loop(0, n_batches // 2, pair_body, 0)
        drain_out(0)
        drain_out(1)

    return k(key_flat, table_half, pe_half)


def _tc_linear_t(xa, xb, wa, wb, b_col, M, D, B):
    nt = (((1,), (1,)), ((), ()))

    def body(xa_ref, xb_ref, wa_ref, wb_ref, b_ref, o_ref):
        slab = lax.dot_general(wa_ref[...], xa_ref[...], nt,
                               preferred_element_type=jnp.float32)
        slab = slab + lax.dot_general(wb_ref[...], xb_ref[...], nt,
                                      preferred_element_type=jnp.float32)
        o_ref[0] = slab + b_ref[...]

    Dh = D // 2
    return pl.pallas_call(
        body,
        grid=(M,),
        in_specs=[
            pl.BlockSpec((B, Dh), lambda i: (i, 0)),
            pl.BlockSpec((B, Dh), lambda i: (i, 0)),
            pl.BlockSpec((D, Dh), lambda i: (0, 0)),
            pl.BlockSpec((D, Dh), lambda i: (0, 0)),
            pl.BlockSpec((D, 1), lambda i: (0, 0)),
        ],
        out_specs=pl.BlockSpec((1, D, B), lambda i: (i, 0, 0)),
        out_shape=jax.ShapeDtypeStruct((M, D, B), jnp.float32),
    )(xa, xb, wa, wb, b_col)


def kernel(key, embedding_table, pe, A_w, A_b):
    B, M, L = key.shape
    V, D = embedding_table.shape
    S = B * M
    Dh = D // 2
    perm32 = np.concatenate([np.arange(0, 32, 2), np.arange(1, 32, 2)])
    P32 = np.zeros((32, 32), dtype=np.float32)
    P32[perm32, np.arange(32)] = 1.0

    key_flat = key.reshape(S * L).astype(jnp.int32)
    halves = []
    weights = []
    for h in range(2):
        cols = slice(h * Dh, (h + 1) * Dh)
        summed_h = _sc_weighted_segsum(
            key_flat,
            embedding_table[:, cols].astype(jnp.bfloat16),
            jnp.dot(pe[:, cols], P32),
            B, M, L,
        )
        halves.append(summed_h)
        weights.append(jnp.dot(A_w[:, cols], P32))

    y_t = _tc_linear_t(halves[0], halves[1], weights[0], weights[1],
                       A_b.reshape(D, 1), M, D, B)
    return jnp.transpose(y_t, (2, 0, 1))


# speedup vs baseline: 1.2689x; 1.2689x over previous
import functools

import jax
import jax.numpy as jnp
import numpy as np
from jax import lax
from jax.experimental import pallas as pl
from jax.experimental.pallas import tpu as pltpu
from jax.experimental.pallas import tpu_sc as plsc

NC = 2
NS = 16
NW = NC * NS
LANES = 16

SEG_BATCH = 32


def _unpack_perm(D):
    parts = []
    for h in range(D // 32):
        base = h * 32
        parts.append(np.arange(base, base + 32, 2))
        parts.append(np.arange(base + 1, base + 32, 2))
    return np.concatenate(parts)


def _sc_weighted_segsum(key_flat, table_bf16, pe_perm, B, M, L, D):
    S = B * M
    segs_per_w = S // NW
    n_batches = segs_per_w // SEG_BATCH
    rows_per_batch = SEG_BATCH * L
    idx_chunks = rows_per_batch // 128
    idx_per_w = n_batches * idx_chunks * 128

    mesh = plsc.VectorSubcoreMesh(core_axis_name="c", subcore_axis_name="s")

    @functools.partial(
        pl.kernel,
        out_type=jax.ShapeDtypeStruct((S, D), jnp.float32),
        mesh=mesh,
        scratch_types=[
            pltpu.VMEM((idx_per_w,), jnp.int32),
            pltpu.VMEM((L, D), jnp.float32),
            pltpu.VMEM((2, rows_per_batch, D), jnp.bfloat16),
            pltpu.VMEM((2, SEG_BATCH, D), jnp.float32),
            pltpu.VMEM((2, SEG_BATCH), jnp.int32),
            pltpu.SemaphoreType.DMA,
            pltpu.SemaphoreType.DMA,
            pltpu.SemaphoreType.DMA,
            pltpu.SemaphoreType.DMA,
        ],
        compiler_params=pltpu.CompilerParams(
            use_tc_tiling_on_sc=False, needs_layout_passes=False
        ),
    )
    def k(key_hbm, table_hbm, pe_hbm, out_hbm, idx_v, pe_v, rows_v, out_v,
          oidx_v, sem0, sem1, osem0, osem1):
        wid = lax.axis_index("s") * NC + lax.axis_index("c")
        pltpu.sync_copy(key_hbm.at[pl.ds(wid * idx_per_w, idx_per_w)], idx_v)
        pltpu.sync_copy(pe_hbm, pe_v)
        sems = (sem0, sem1)
        osems = (osem0, osem1)
        iota16 = lax.iota(jnp.int32, LANES)

        def fire(b, slot):
            for j in range(idx_chunks):
                pltpu.async_copy(
                    table_hbm.at[idx_v.at[pl.ds((b * idx_chunks + j) * 128, 128)]],
                    rows_v.at[slot].at[pl.ds(j * 128, 128)],
                    sems[slot],
                )

        def drain(slot):
            pltpu.make_async_copy(
                table_hbm.at[pl.ds(0, rows_per_batch)],
                rows_v.at[slot],
                sems[slot],
            ).wait()

        def drain_out(slot):
            pltpu.make_async_copy(
                out_v.at[slot],
                out_hbm.at[pl.ds(0, SEG_BATCH)],
                osems[slot],
            ).wait()

        def compute(b, slot):
            @pl.when(b >= 2)
            def _(slot=slot):
                drain_out(slot)

            s0 = wid * segs_per_w + b * SEG_BATCH
            for h in range(SEG_BATCH // LANES):
                sv = iota16 + (s0 + h * LANES)
                rv = (sv % M) * B + sv // M
                oidx_v[slot, pl.ds(h * LANES, LANES)] = rv

            for c in range(D // 32):
                sl32 = pl.ds(c * 32, 32)
                pe_e = [pe_v[l, pl.ds(c * 32, LANES)] for l in range(L)]
                pe_o = [pe_v[l, pl.ds(c * 32 + LANES, LANES)] for l in range(L)]

                def seg_body(s, _, sl32=sl32, pe_e=pe_e, pe_o=pe_o, slot=slot,
                             c=c):
                    base = s * L
                    packed = rows_v[slot, base, sl32]
                    ev, od = plsc.unpack(
                        packed,
                        format=plsc.PackFormat.INTERLEAVED,
                        preferred_element_type=jnp.float32,
                    )
                    acc_e = pe_e[0] * ev
                    acc_o = pe_o[0] * od
                    for l in range(1, L):
                        packed = rows_v[slot, base + l, sl32]
                        ev, od = plsc.unpack(
                            packed,
                            format=plsc.PackFormat.INTERLEAVED,
                            preferred_element_type=jnp.float32,
                        )
                        acc_e = acc_e + pe_e[l] * ev
                        acc_o = acc_o + pe_o[l] * od
                    out_v[slot, s, pl.ds(c * 32, LANES)] = acc_e
                    out_v[slot, s, pl.ds(c * 32 + LANES, LANES)] = acc_o
                    return 0

                lax.fori_loop(0, SEG_BATCH, seg_body, 0)

            pltpu.async_copy(
                out_v.at[slot], out_hbm.at[oidx_v.at[slot]], osems[slot]
            )

        fire(0, 0)
        fire(1, 1)

        def pair_body(i, carry):
            b = i * 2
            for slot in range(2):
                drain(slot)
                compute(b + slot, slot)

                @pl.when(b + slot + 2 < n_batches)
                def _(b=b, slot=slot):
                    fire(b + slot + 2, slot)

            return carry

        lax.fori_loop(0, n_batches // 2, pair_body, 0)
        drain_out(0)
        drain_out(1)

    return k(key_flat, table_bf16, pe_perm)


def _tc_linear_t(x, w2, b_col, M, D, B):
    def body(x_ref, w_ref, b_ref, o_ref):
        slab = lax.dot_general(
            w_ref[...], x_ref[...],
            dimension_numbers=(((1,), (1,)), ((), ())),
            preferred_element_type=jnp.float32,
        )
        o_ref[0] = slab + b_ref[...]

    return pl.pallas_call(
        body,
        grid=(M,),
        in_specs=[
            pl.BlockSpec((B, D), lambda i: (i, 0)),
            pl.BlockSpec((D, D), lambda i: (0, 0)),
            pl.BlockSpec((D, 1), lambda i: (0, 0)),
        ],
        out_specs=pl.BlockSpec((1, D, B), lambda i: (i, 0, 0)),
        out_shape=jax.ShapeDtypeStruct((M, D, B), jnp.float32),
    )(x, w2, b_col)


def kernel(key, embedding_table, pe, A_w, A_b):
    B, M, L = key.shape
    V, D = embedding_table.shape
    S = B * M
    perm = _unpack_perm(D)
    P = np.zeros((D, D), dtype=np.float32)
    P[perm, np.arange(D)] = 1.0
    summed = _sc_weighted_segsum(
        key.reshape(S * L).astype(jnp.int32),
        embedding_table.astype(jnp.bfloat16),
        jnp.dot(pe, P),
        B, M, L, D,
    )
    y_t = _tc_linear_t(summed, jnp.dot(A_w, P), A_b.reshape(D, 1), M, D, B)
    return jnp.transpose(y_t, (2, 0, 1))


# speedup vs baseline: 1.3640x; 1.0749x over previous
import functools

import jax
import jax.numpy as jnp
import numpy as np
from jax import lax
from jax.experimental import pallas as pl
from jax.experimental.pallas import tpu as pltpu
from jax.experimental.pallas import tpu_sc as plsc

NC = 2
NS = 16
NW = NC * NS
LANES = 16

SEG_BATCH = 32


def _unpack_perm(D):
    parts = []
    for h in range(D // 32):
        base = h * 32
        parts.append(np.arange(base, base + 32, 2))
        parts.append(np.arange(base + 1, base + 32, 2))
    return np.concatenate(parts)


def _sc_weighted_segsum(key_flat, table_bf16, pe_perm, B, M, L, D):
    S = B * M
    segs_per_w = S // NW
    n_batches = segs_per_w // SEG_BATCH
    rows_per_batch = SEG_BATCH * L
    idx_chunks = rows_per_batch // 128
    idx_per_w = n_batches * idx_chunks * 128

    mesh = plsc.VectorSubcoreMesh(core_axis_name="c", subcore_axis_name="s")

    @functools.partial(
        pl.kernel,
        out_type=jax.ShapeDtypeStruct((S, D), jnp.float32),
        mesh=mesh,
        scratch_types=[
            pltpu.VMEM((idx_per_w,), jnp.int32),
            pltpu.VMEM((L, D), jnp.float32),
            pltpu.VMEM((2, rows_per_batch, D), jnp.bfloat16),
            pltpu.VMEM((2, SEG_BATCH, D), jnp.float32),
            pltpu.VMEM((2, SEG_BATCH), jnp.int32),
            pltpu.SemaphoreType.DMA,
            pltpu.SemaphoreType.DMA,
            pltpu.SemaphoreType.DMA,
            pltpu.SemaphoreType.DMA,
        ],
        compiler_params=pltpu.CompilerParams(
            use_tc_tiling_on_sc=False, needs_layout_passes=False
        ),
    )
    def k(key_hbm, table_hbm, pe_hbm, out_hbm, idx_v, pe_v, rows_v, out_v,
          oidx_v, sem0, sem1, osem0, osem1):
        wid = lax.axis_index("s") * NC + lax.axis_index("c")
        pltpu.sync_copy(key_hbm.at[pl.ds(wid * idx_per_w, idx_per_w)], idx_v)
        pltpu.sync_copy(pe_hbm, pe_v)
        sems = (sem0, sem1)
        osems = (osem0, osem1)
        iota16 = lax.iota(jnp.int32, LANES)

        def fire(b, slot):
            for j in range(idx_chunks):
                pltpu.async_copy(
                    table_hbm.at[idx_v.at[pl.ds((b * idx_chunks + j) * 128, 128)]],
                    rows_v.at[slot].at[pl.ds(j * 128, 128)],
                    sems[slot],
                )

        def drain(slot):
            pltpu.make_async_copy(
                table_hbm.at[pl.ds(0, rows_per_batch)],
                rows_v.at[slot],
                sems[slot],
            ).wait()

        def drain_out(slot):
            pltpu.make_async_copy(
                out_v.at[slot],
                out_hbm.at[pl.ds(0, SEG_BATCH)],
                osems[slot],
            ).wait()

        def compute(b, slot):
            @pl.when(b >= 2)
            def _(slot=slot):
                drain_out(slot)

            s0 = wid * segs_per_w + b * SEG_BATCH
            for h in range(SEG_BATCH // LANES):
                sv = iota16 + (s0 + h * LANES)
                rv = (sv % M) * B + sv // M
                oidx_v[slot, pl.ds(h * LANES, LANES)] = rv

            for c in range(D // 32):
                sl32 = pl.ds(c * 32, 32)
                pe_e = [pe_v[l, pl.ds(c * 32, LANES)] for l in range(L)]
                pe_o = [pe_v[l, pl.ds(c * 32 + LANES, LANES)] for l in range(L)]

                def seg_body(s, _, sl32=sl32, pe_e=pe_e, pe_o=pe_o, slot=slot,
                             c=c):
                    base = s * L
                    packed = rows_v[slot, base, sl32]
                    ev, od = plsc.unpack(
                        packed,
                        format=plsc.PackFormat.INTERLEAVED,
                        preferred_element_type=jnp.float32,
                    )
                    acc_e = pe_e[0] * ev
                    acc_o = pe_o[0] * od
                    for l in range(1, L):
                        packed = rows_v[slot, base + l, sl32]
                        ev, od = plsc.unpack(
                            packed,
                            format=plsc.PackFormat.INTERLEAVED,
                            preferred_element_type=jnp.float32,
                        )
                        acc_e = acc_e + pe_e[l] * ev
                        acc_o = acc_o + pe_o[l] * od
                    out_v[slot, s, pl.ds(c * 32, LANES)] = acc_e
                    out_v[slot, s, pl.ds(c * 32 + LANES, LANES)] = acc_o
                    return 0

                lax.fori_loop(0, SEG_BATCH, seg_body, 0)

            pltpu.async_copy(
                out_v.at[slot], out_hbm.at[oidx_v.at[slot]], osems[slot]
            )

        fire(0, 0)
        fire(1, 1)

        def pair_body(i, carry):
            b = i * 2
            for slot in range(2):
                drain(slot)
                compute(b + slot, slot)

                @pl.when(b + slot + 2 < n_batches)
                def _(b=b, slot=slot):
                    fire(b + slot + 2, slot)

            return carry

        lax.fori_loop(0, n_batches // 2, pair_body, 0)
        drain_out(0)
        drain_out(1)

    return k(key_flat, table_bf16, pe_perm)


def _tc_linear_t(x, w2, b_col, M, D, B):
    MB = 5

    def body(x_ref, w_ref, b_ref, o_ref):
        for j in range(MB):
            slab = lax.dot_general(
                w_ref[...], x_ref[pl.ds(j * B, B), :],
                dimension_numbers=(((1,), (1,)), ((), ())),
                preferred_element_type=jnp.float32,
            )
            o_ref[j] = slab + b_ref[...]

    return pl.pallas_call(
        body,
        grid=(M // MB,),
        in_specs=[
            pl.BlockSpec((MB * B, D), lambda i: (i, 0)),
            pl.BlockSpec((D, D), lambda i: (0, 0)),
            pl.BlockSpec((D, 1), lambda i: (0, 0)),
        ],
        out_specs=pl.BlockSpec((MB, D, B), lambda i: (i, 0, 0)),
        out_shape=jax.ShapeDtypeStruct((M, D, B), jnp.float32),
    )(x, w2, b_col)


def kernel(key, embedding_table, pe, A_w, A_b):
    B, M, L = key.shape
    V, D = embedding_table.shape
    S = B * M
    perm = _unpack_perm(D)
    P = np.zeros((D, D), dtype=np.float32)
    P[perm, np.arange(D)] = 1.0
    summed = _sc_weighted_segsum(
        key.reshape(S * L).astype(jnp.int32),
        embedding_table.astype(jnp.bfloat16),
        jnp.dot(pe, P),
        B, M, L, D,
    )
    y_t = _tc_linear_t(summed, jnp.dot(A_w, P), A_b.reshape(D, 1), M, D, B)
    return jnp.transpose(y_t, (2, 0, 1))


# speedup vs baseline: 1.3663x; 1.0017x over previous
import functools

import jax
import jax.numpy as jnp
import numpy as np
from jax import lax
from jax.experimental import pallas as pl
from jax.experimental.pallas import tpu as pltpu
from jax.experimental.pallas import tpu_sc as plsc

NC = 2
NS = 16
NW = NC * NS
LANES = 16

SEG_BATCH = 32


def _unpack_perm(D):
    parts = []
    for h in range(D // 32):
        base = h * 32
        parts.append(np.arange(base, base + 32, 2))
        parts.append(np.arange(base + 1, base + 32, 2))
    return np.concatenate(parts)


def _sc_weighted_segsum(key_flat, table_bf16, pe_perm, B, M, L, D):
    S = B * M
    segs_per_w = S // NW
    n_batches = segs_per_w // SEG_BATCH
    rows_per_batch = SEG_BATCH * L
    idx_chunks = rows_per_batch // 128
    idx_per_w = n_batches * idx_chunks * 128

    mesh = plsc.VectorSubcoreMesh(core_axis_name="c", subcore_axis_name="s")

    @functools.partial(
        pl.kernel,
        out_type=jax.ShapeDtypeStruct((S, D), jnp.float32),
        mesh=mesh,
        scratch_types=[
            pltpu.VMEM((idx_per_w,), jnp.int32),
            pltpu.VMEM((L, D), jnp.float32),
            pltpu.VMEM((2, rows_per_batch, D), jnp.bfloat16),
            pltpu.VMEM((2, SEG_BATCH, D), jnp.float32),
            pltpu.VMEM((2, SEG_BATCH), jnp.int32),
            pltpu.SemaphoreType.DMA,
            pltpu.SemaphoreType.DMA,
            pltpu.SemaphoreType.DMA,
            pltpu.SemaphoreType.DMA,
        ],
        compiler_params=pltpu.CompilerParams(
            use_tc_tiling_on_sc=False, needs_layout_passes=False
        ),
    )
    def k(key_hbm, table_hbm, pe_hbm, out_hbm, idx_v, pe_v, rows_v, out_v,
          oidx_v, sem0, sem1, osem0, osem1):
        wid = lax.axis_index("s") * NC + lax.axis_index("c")
        pltpu.sync_copy(key_hbm.at[pl.ds(wid * idx_per_w, idx_per_w)], idx_v)
        pltpu.sync_copy(pe_hbm, pe_v)
        sems = (sem0, sem1)
        osems = (osem0, osem1)
        iota16 = lax.iota(jnp.int32, LANES)

        def fire(b, slot):
            for j in range(idx_chunks):
                pltpu.async_copy(
                    table_hbm.at[idx_v.at[pl.ds((b * idx_chunks + j) * 128, 128)]],
                    rows_v.at[slot].at[pl.ds(j * 128, 128)],
                    sems[slot],
                )

        def drain(slot):
            pltpu.make_async_copy(
                table_hbm.at[pl.ds(0, rows_per_batch)],
                rows_v.at[slot],
                sems[slot],
            ).wait()

        def drain_out(slot):
            pltpu.make_async_copy(
                out_v.at[slot],
                out_hbm.at[pl.ds(0, SEG_BATCH)],
                osems[slot],
            ).wait()

        def compute(b, slot):
            @pl.when(b >= 2)
            def _(slot=slot):
                drain_out(slot)

            s0 = wid * segs_per_w + b * SEG_BATCH
            for h in range(SEG_BATCH // LANES):
                sv = iota16 + (s0 + h * LANES)
                rv = (sv % M) * B + sv // M
                oidx_v[slot, pl.ds(h * LANES, LANES)] = rv

            for c in range(D // 32):
                sl32 = pl.ds(c * 32, 32)
                pe_e = [pe_v[l, pl.ds(c * 32, LANES)] for l in range(L)]
                pe_o = [pe_v[l, pl.ds(c * 32 + LANES, LANES)] for l in range(L)]

                def seg_body(s, _, sl32=sl32, pe_e=pe_e, pe_o=pe_o, slot=slot,
                             c=c):
                    base = s * L
                    packed = rows_v[slot, base, sl32]
                    ev, od = plsc.unpack(
                        packed,
                        format=plsc.PackFormat.INTERLEAVED,
                        preferred_element_type=jnp.float32,
                    )
                    acc_e = pe_e[0] * ev
                    acc_o = pe_o[0] * od
                    for l in range(1, L):
                        packed = rows_v[slot, base + l, sl32]
                        ev, od = plsc.unpack(
                            packed,
                            format=plsc.PackFormat.INTERLEAVED,
                            preferred_element_type=jnp.float32,
                        )
                        acc_e = acc_e + pe_e[l] * ev
                        acc_o = acc_o + pe_o[l] * od
                    out_v[slot, s, pl.ds(c * 32, LANES)] = acc_e
                    out_v[slot, s, pl.ds(c * 32 + LANES, LANES)] = acc_o
                    return 0

                lax.fori_loop(0, SEG_BATCH, seg_body, 0)

            pltpu.async_copy(
                out_v.at[slot], out_hbm.at[oidx_v.at[slot]], osems[slot]
            )

        fire(0, 0)
        fire(1, 1)

        def pair_body(i, carry):
            b = i * 2
            for slot in range(2):
                drain(slot)
                compute(b + slot, slot)

                @pl.when(b + slot + 2 < n_batches)
                def _(b=b, slot=slot):
                    fire(b + slot + 2, slot)

            return carry

        lax.fori_loop(0, n_batches // 2, pair_body, 0)
        drain_out(0)
        drain_out(1)

    return k(key_flat, table_bf16, pe_perm)


def _tc_linear_t(x, w2, b_col, M, D, B):
    MB = 10

    def body(x_ref, w_ref, b_ref, o_ref):
        for j in range(MB):
            slab = lax.dot_general(
                w_ref[...], x_ref[pl.ds(j * B, B), :],
                dimension_numbers=(((1,), (1,)), ((), ())),
                preferred_element_type=jnp.float32,
            )
            o_ref[j] = slab + b_ref[...]

    return pl.pallas_call(
        body,
        grid=(M // MB,),
        in_specs=[
            pl.BlockSpec((MB * B, D), lambda i: (i, 0)),
            pl.BlockSpec((D, D), lambda i: (0, 0)),
            pl.BlockSpec((D, 1), lambda i: (0, 0)),
        ],
        out_specs=pl.BlockSpec((MB, D, B), lambda i: (i, 0, 0)),
        out_shape=jax.ShapeDtypeStruct((M, D, B), jnp.float32),
    )(x, w2, b_col)


def kernel(key, embedding_table, pe, A_w, A_b):
    B, M, L = key.shape
    V, D = embedding_table.shape
    S = B * M
    perm = _unpack_perm(D)
    P = np.zeros((D, D), dtype=np.float32)
    P[perm, np.arange(D)] = 1.0
    summed = _sc_weighted_segsum(
        key.reshape(S * L).astype(jnp.int32),
        embedding_table.astype(jnp.bfloat16),
        jnp.dot(pe, P),
        B, M, L, D,
    )
    y_t = _tc_linear_t(summed, jnp.dot(A_w, P), A_b.reshape(D, 1), M, D, B)
    return jnp.transpose(y_t, (2, 0, 1))
